# Initial kernel scaffold; baseline (speedup 1.0000x reference)
import jax
import jax.numpy as jnp
from jax.experimental import pallas as pl


def kernel(x, edge_index, edge_attr, W1, b1, W2, b2, ln_gamma, ln_beta):
    raise NotImplementedError("write your pallas kernel here")


# speedup vs baseline: 6.7175x; 6.7175x over previous
import functools

import jax
import jax.numpy as jnp
from jax import lax
from jax.experimental import pallas as pl
from jax.experimental.pallas import tpu as pltpu
from jax.experimental.pallas import tpu_sc as plsc

NC = 2
NS = 16
NW = NC * NS
CHUNK = 128
SG = 10
KPT = 80


def _sc_scatter_add(attr1, idx3d, zeros, n_nodes, n_chunks):
    mesh = plsc.VectorSubcoreMesh(core_axis_name="c", subcore_axis_name="s")
    rows_per_tile = n_nodes // NS
    assert n_nodes % (NS * 8) == 0
    full_tiles = n_chunks // KPT
    rem_chunks = n_chunks - full_tiles * KPT
    sgw = SG * 8 * CHUNK

    @functools.partial(
        pl.kernel,
        mesh=mesh,
        compiler_params=pltpu.CompilerParams(
            use_tc_tiling_on_sc=False, needs_layout_passes=False),
        out_type=jax.ShapeDtypeStruct((NC, n_nodes, 16), jnp.float32),
        scratch_types=[
            pltpu.VMEM((KPT, CHUNK), jnp.int32),
            pltpu.VMEM((2 * (SG * 8 + 8), CHUNK + 1), jnp.float32),
            pltpu.VMEM((CHUNK, 16), jnp.float32),
            pltpu.VMEM_SHARED((n_nodes, 16), jnp.float32),
        ],
    )
    def sc_kernel(attr_hbm, idx_hbm, zeros_hbm, out_hbm, idx_v, raw_v,
                  tch_v, accum):
        c = lax.axis_index("c")
        s = lax.axis_index("s")
        wid = s * NC + c

        pltpu.sync_copy(
            zeros_hbm.at[pl.ds(s * rows_per_tile, rows_per_tile)],
            accum.at[pl.ds(s * rows_per_tile, rows_per_tile)],
        )
        pltpu.sync_copy(idx_hbm.at[wid], idx_v)
        plsc.subcore_barrier()

        rpp = SG * 8 + 8
        lanes = lax.iota(jnp.int32, 16)
        rvec = (lanes // 8) * rpp + (lanes % 8)

        nsg = jnp.where(wid < full_tiles, KPT // SG, rem_chunks // SG)

        @pl.loop(0, nsg)
        def _(g):
            c0 = wid * KPT + g * SG
            for tr in range(2):
                pltpu.sync_copy(
                    attr_hbm.at[pl.ds((tr * n_chunks + c0) * 8, SG * 8)],
                    raw_v.at[pl.ds(tr * rpp, SG * 8), pl.ds(0, CHUNK)])
            for j in range(SG):
                rows_j = rvec + j * 8

                @pl.loop(0, CHUNK // 16)
                def _(it):
                    e0 = it * 16
                    for u in range(16):
                        ev = jnp.full((16,), 0, jnp.int32) + (e0 + u)
                        row = plsc.load_gather(raw_v, [rows_j, ev])
                        tch_v[e0 + u, :] = row

                pltpu.sync_copy(
                    tch_v,
                    accum.at[idx_v.at[g * SG + j]],
                    add=True,
                )

        plsc.subcore_barrier()
        pltpu.sync_copy(
            accum.at[pl.ds(s * rows_per_tile, rows_per_tile)],
            out_hbm.at[c].at[pl.ds(s * rows_per_tile, rows_per_tile)],
        )

    return sc_kernel(attr1, idx3d, zeros)


def _tc_mlp_body(x_ref, p_ref, w1x_ref, w1a_ref, b1_ref, w2_ref, b2_ref,
                 g_ref, bt_ref, o_ref):
    x = x_ref[...]
    agg = p_ref[0] + p_ref[1]
    h = jnp.dot(x, w1x_ref[...], preferred_element_type=jnp.float32)
    h = h + jnp.dot(agg, w1a_ref[...], preferred_element_type=jnp.float32)
    h = h + b1_ref[...]
    h = h * jax.nn.sigmoid(h)
    h = jnp.dot(h, w2_ref[...], preferred_element_type=jnp.float32) + b2_ref[...]
    mean = jnp.mean(h, axis=-1, keepdims=True)
    hc = h - mean
    var = jnp.mean(hc * hc, axis=-1, keepdims=True)
    h = hc * lax.rsqrt(var + 1e-5) * g_ref[...] + bt_ref[...]
    o_ref[...] = x + h


def _tc_mlp(x, partials, W1, b1, W2, b2, ln_gamma, ln_beta, block_rows):
    n, d_feat = x.shape
    d_edge = partials.shape[-1]
    d_hid = W1.shape[1]
    d_out = W2.shape[1]
    grid = (n // block_rows,)
    full = lambda shape: pl.BlockSpec(shape, lambda i: (0,) * len(shape))
    return pl.pallas_call(
        _tc_mlp_body,
        grid=grid,
        in_specs=[
            pl.BlockSpec((block_rows, d_feat), lambda i: (i, 0)),
            pl.BlockSpec((NC, block_rows, d_edge), lambda i: (0, i, 0)),
            full((d_feat, d_hid)),
            full((d_edge, d_hid)),
            full((1, d_hid)),
            full((d_hid, d_out)),
            full((1, d_out)),
            full((1, d_out)),
            full((1, d_out)),
        ],
        out_specs=pl.BlockSpec((block_rows, d_out), lambda i: (i, 0)),
        out_shape=jax.ShapeDtypeStruct((n, d_out), jnp.float32),
    )(x, partials, W1[:d_feat], W1[d_feat:], b1.reshape(1, -1), W2,
      b2.reshape(1, -1), ln_gamma.reshape(1, -1), ln_beta.reshape(1, -1))


def kernel(x, edge_index, edge_attr, W1, b1, W2, b2, ln_gamma, ln_beta):
    n, d_feat = x.shape
    e, d_edge = edge_attr.shape

    assert d_edge == 16 and e % CHUNK == 0
    n_chunks = e // CHUNK
    assert n_chunks % SG == 0

    n_pad = -(-n // (NS * 8)) * (NS * 8)

    dst = edge_index[1].astype(jnp.int32)
    e_pad = NW * KPT * CHUNK
    idx3d = jnp.zeros((e_pad,), jnp.int32).at[:e].set(dst).reshape(
        NW, KPT, CHUNK)
    attr1 = edge_attr.astype(jnp.float32).T.reshape(
        2, 8, n_chunks, CHUNK).transpose(0, 2, 1, 3).reshape(-1).reshape(
        2 * n_chunks * 8, CHUNK)
    zeros = jnp.zeros((n_pad, d_edge), jnp.float32)

    partials = _sc_scatter_add(attr1, idx3d, zeros, n_pad, n_chunks)[:, :n]
    return _tc_mlp(x, partials, W1, b1, W2, b2, ln_gamma, ln_beta,
                   block_rows=2000)


# speedup vs baseline: 8.3401x; 1.2415x over previous
import functools

import jax
import jax.numpy as jnp
from jax import lax
from jax.experimental import pallas as pl
from jax.experimental.pallas import tpu as pltpu
from jax.experimental.pallas import tpu_sc as plsc

NC = 2
NS = 16
NW = NC * NS
CHUNK = 128
SG = 10
KPT = 80


def _sc_scatter_add(attr1, idx3d, zeros, n_nodes, n_chunks):
    mesh = plsc.VectorSubcoreMesh(core_axis_name="c", subcore_axis_name="s")
    rows_per_tile = n_nodes // NS
    assert n_nodes % (NS * 8) == 0
    full_tiles = n_chunks // KPT
    rem_chunks = n_chunks - full_tiles * KPT
    sgw = SG * 8 * CHUNK

    @functools.partial(
        pl.kernel,
        mesh=mesh,
        compiler_params=pltpu.CompilerParams(
            use_tc_tiling_on_sc=False, needs_layout_passes=False),
        out_type=jax.ShapeDtypeStruct((NC, n_nodes, 16), jnp.float32),
        scratch_types=[
            pltpu.VMEM((KPT, CHUNK), jnp.int32),
            pltpu.VMEM((2 * (SG * 8 + 8), CHUNK + 1), jnp.float32),
            pltpu.VMEM((CHUNK, 16), jnp.float32),
            pltpu.VMEM_SHARED((n_nodes, 16), jnp.float32),
        ],
    )
    def sc_kernel(attr_hbm, idx_hbm, zeros_hbm, out_hbm, idx_v, raw_v,
                  tch_v, accum):
        c = lax.axis_index("c")
        s = lax.axis_index("s")
        wid = s * NC + c

        pltpu.sync_copy(
            zeros_hbm.at[pl.ds(s * rows_per_tile, rows_per_tile)],
            accum.at[pl.ds(s * rows_per_tile, rows_per_tile)],
        )
        pltpu.sync_copy(idx_hbm.at[wid], idx_v)
        plsc.subcore_barrier()

        rpp = SG * 8 + 8
        lanes = lax.iota(jnp.int32, 16)
        rvec = (lanes // 8) * rpp + (lanes % 8)

        nsg = jnp.where(wid < full_tiles, KPT // SG, rem_chunks // SG)

        @pl.loop(0, nsg)
        def _(g):
            c0 = wid * KPT + g * SG
            for tr in range(2):
                pltpu.sync_copy(
                    attr_hbm.at[pl.ds((tr * n_chunks + c0) * 8, SG * 8)],
                    raw_v.at[pl.ds(tr * rpp, SG * 8), pl.ds(0, CHUNK)])
            for j in range(SG):
                rows_j = rvec + j * 8

                @plsc.parallel_loop(0, CHUNK, step=16, unroll=2)
                def _(e0):
                    for u in range(16):
                        ev = jnp.full((16,), 0, jnp.int32) + (e0 + u)
                        row = plsc.load_gather(raw_v, [rows_j, ev])
                        tch_v[e0 + u, :] = row

                pltpu.sync_copy(
                    tch_v,
                    accum.at[idx_v.at[g * SG + j]],
                    add=True,
                )

        plsc.subcore_barrier()
        pltpu.sync_copy(
            accum.at[pl.ds(s * rows_per_tile, rows_per_tile)],
            out_hbm.at[c].at[pl.ds(s * rows_per_tile, rows_per_tile)],
        )

    return sc_kernel(attr1, idx3d, zeros)


def _tc_mlp_body(x_ref, p_ref, w1x_ref, w1a_ref, b1_ref, w2_ref, b2_ref,
                 g_ref, bt_ref, o_ref):
    x = x_ref[...]
    agg = p_ref[0] + p_ref[1]
    h = jnp.dot(x, w1x_ref[...], preferred_element_type=jnp.float32)
    h = h + jnp.dot(agg, w1a_ref[...], preferred_element_type=jnp.float32)
    h = h + b1_ref[...]
    h = h * jax.nn.sigmoid(h)
    h = jnp.dot(h, w2_ref[...], preferred_element_type=jnp.float32) + b2_ref[...]
    mean = jnp.mean(h, axis=-1, keepdims=True)
    hc = h - mean
    var = jnp.mean(hc * hc, axis=-1, keepdims=True)
    h = hc * lax.rsqrt(var + 1e-5) * g_ref[...] + bt_ref[...]
    o_ref[...] = x + h


def _tc_mlp(x, partials, W1, b1, W2, b2, ln_gamma, ln_beta, block_rows):
    n, d_feat = x.shape
    d_edge = partials.shape[-1]
    d_hid = W1.shape[1]
    d_out = W2.shape[1]
    grid = (n // block_rows,)
    full = lambda shape: pl.BlockSpec(shape, lambda i: (0,) * len(shape))
    return pl.pallas_call(
        _tc_mlp_body,
        grid=grid,
        in_specs=[
            pl.BlockSpec((block_rows, d_feat), lambda i: (i, 0)),
            pl.BlockSpec((NC, block_rows, d_edge), lambda i: (0, i, 0)),
            full((d_feat, d_hid)),
            full((d_edge, d_hid)),
            full((1, d_hid)),
            full((d_hid, d_out)),
            full((1, d_out)),
            full((1, d_out)),
            full((1, d_out)),
        ],
        out_specs=pl.BlockSpec((block_rows, d_out), lambda i: (i, 0)),
        out_shape=jax.ShapeDtypeStruct((n, d_out), jnp.float32),
    )(x, partials, W1[:d_feat], W1[d_feat:], b1.reshape(1, -1), W2,
      b2.reshape(1, -1), ln_gamma.reshape(1, -1), ln_beta.reshape(1, -1))


def kernel(x, edge_index, edge_attr, W1, b1, W2, b2, ln_gamma, ln_beta):
    n, d_feat = x.shape
    e, d_edge = edge_attr.shape

    assert d_edge == 16 and e % CHUNK == 0
    n_chunks = e // CHUNK
    assert n_chunks % SG == 0

    n_pad = -(-n // (NS * 8)) * (NS * 8)

    dst = edge_index[1].astype(jnp.int32)
    e_pad = NW * KPT * CHUNK
    idx3d = jnp.zeros((e_pad,), jnp.int32).at[:e].set(dst).reshape(
        NW, KPT, CHUNK)
    attr1 = edge_attr.astype(jnp.float32).T.reshape(
        2, 8, n_chunks, CHUNK).transpose(0, 2, 1, 3).reshape(-1).reshape(
        2 * n_chunks * 8, CHUNK)
    zeros = jnp.zeros((n_pad, d_edge), jnp.float32)

    partials = _sc_scatter_add(attr1, idx3d, zeros, n_pad, n_chunks)[:, :n]
    return _tc_mlp(x, partials, W1, b1, W2, b2, ln_gamma, ln_beta,
                   block_rows=2000)


# speedup vs baseline: 9.1205x; 1.0936x over previous
import functools

import jax
import jax.numpy as jnp
from jax import lax
from jax.experimental import pallas as pl
from jax.experimental.pallas import tpu as pltpu
from jax.experimental.pallas import tpu_sc as plsc

NC = 2
NS = 16
NW = NC * NS
CHUNK = 128
SG = 10
KPT = 80


def _sc_scatter_add(attr1, idx3d, zeros, n_nodes, n_chunks):
    mesh = plsc.VectorSubcoreMesh(core_axis_name="c", subcore_axis_name="s")
    rows_per_tile = n_nodes // NS
    assert n_nodes % (NS * 8) == 0
    full_tiles = n_chunks // KPT
    rem_chunks = n_chunks - full_tiles * KPT
    sgw = SG * 8 * CHUNK

    @functools.partial(
        pl.kernel,
        mesh=mesh,
        compiler_params=pltpu.CompilerParams(
            use_tc_tiling_on_sc=False, needs_layout_passes=False),
        out_type=jax.ShapeDtypeStruct((NC, n_nodes, 16), jnp.float32),
        scratch_types=[
            pltpu.VMEM((KPT, CHUNK), jnp.int32),
            pltpu.VMEM((2 * (SG * 8 + 8), CHUNK + 1), jnp.float32),
            pltpu.VMEM((CHUNK, 16), jnp.float32),
            pltpu.VMEM_SHARED((n_nodes, 16), jnp.float32),
        ],
    )
    def sc_kernel(attr_hbm, idx_hbm, zeros_hbm, out_hbm, idx_v, raw_v,
                  tch_v, accum):
        c = lax.axis_index("c")
        s = lax.axis_index("s")
        wid = s * NC + c

        pltpu.sync_copy(
            zeros_hbm.at[pl.ds(s * rows_per_tile, rows_per_tile)],
            accum.at[pl.ds(s * rows_per_tile, rows_per_tile)],
        )
        @pl.when(wid < full_tiles)
        def _():
            pltpu.sync_copy(idx_hbm.at[1].at[pl.ds(wid * KPT, KPT)], idx_v)

        @pl.when(wid == full_tiles)
        def _():
            pltpu.sync_copy(
                idx_hbm.at[1].at[pl.ds(full_tiles * KPT, rem_chunks)],
                idx_v.at[pl.ds(0, rem_chunks)])

        plsc.subcore_barrier()

        rpp = SG * 8 + 8
        lanes = lax.iota(jnp.int32, 16)
        rvec = (lanes // 8) * rpp + (lanes % 8)

        nsg = jnp.where(wid < full_tiles, KPT // SG, rem_chunks // SG)

        @pl.loop(0, nsg)
        def _(g):
            c0 = wid * KPT + g * SG
            for tr in range(2):
                pltpu.sync_copy(
                    attr_hbm.at[pl.ds((tr * n_chunks + c0) * 8, SG * 8)],
                    raw_v.at[pl.ds(tr * rpp, SG * 8), pl.ds(0, CHUNK)])
            for j in range(SG):
                rows_j = rvec + j * 8

                @plsc.parallel_loop(0, CHUNK, step=16, unroll=2)
                def _(e0):
                    for u in range(16):
                        ev = jnp.full((16,), 0, jnp.int32) + (e0 + u)
                        row = plsc.load_gather(raw_v, [rows_j, ev])
                        tch_v[e0 + u, :] = row

                pltpu.sync_copy(
                    tch_v,
                    accum.at[idx_v.at[g * SG + j]],
                    add=True,
                )

        plsc.subcore_barrier()
        pltpu.sync_copy(
            accum.at[pl.ds(s * rows_per_tile, rows_per_tile)],
            out_hbm.at[c].at[pl.ds(s * rows_per_tile, rows_per_tile)],
        )

    return sc_kernel(attr1, idx3d, zeros)


def _tc_mlp_body(x_ref, p_ref, w1x_ref, w1a_ref, b1_ref, w2_ref, b2_ref,
                 g_ref, bt_ref, o_ref):
    x = x_ref[...]
    agg = p_ref[0] + p_ref[1]
    h = jnp.dot(x, w1x_ref[...], preferred_element_type=jnp.float32)
    h = h + jnp.dot(agg, w1a_ref[...], preferred_element_type=jnp.float32)
    h = h + b1_ref[...]
    h = h * jax.nn.sigmoid(h)
    h = jnp.dot(h, w2_ref[...], preferred_element_type=jnp.float32) + b2_ref[...]
    mean = jnp.mean(h, axis=-1, keepdims=True)
    hc = h - mean
    var = jnp.mean(hc * hc, axis=-1, keepdims=True)
    h = hc * lax.rsqrt(var + 1e-5) * g_ref[...] + bt_ref[...]
    o_ref[...] = x + h


def _tc_mlp(x, partials, W1, b1, W2, b2, ln_gamma, ln_beta, block_rows):
    n, d_feat = x.shape
    d_edge = partials.shape[-1]
    d_hid = W1.shape[1]
    d_out = W2.shape[1]
    grid = (n // block_rows,)
    full = lambda shape: pl.BlockSpec(shape, lambda i: (0,) * len(shape))
    return pl.pallas_call(
        _tc_mlp_body,
        grid=grid,
        in_specs=[
            pl.BlockSpec((block_rows, d_feat), lambda i: (i, 0)),
            pl.BlockSpec((NC, block_rows, d_edge), lambda i: (0, i, 0)),
            full((d_feat, d_hid)),
            full((d_edge, d_hid)),
            full((1, d_hid)),
            full((d_hid, d_out)),
            full((1, d_out)),
            full((1, d_out)),
            full((1, d_out)),
        ],
        out_specs=pl.BlockSpec((block_rows, d_out), lambda i: (i, 0)),
        out_shape=jax.ShapeDtypeStruct((n, d_out), jnp.float32),
    )(x, partials, W1[:d_feat], W1[d_feat:], b1.reshape(1, -1), W2,
      b2.reshape(1, -1), ln_gamma.reshape(1, -1), ln_beta.reshape(1, -1))


def kernel(x, edge_index, edge_attr, W1, b1, W2, b2, ln_gamma, ln_beta):
    n, d_feat = x.shape
    e, d_edge = edge_attr.shape

    assert d_edge == 16 and e % CHUNK == 0
    n_chunks = e // CHUNK
    assert n_chunks % SG == 0

    n_pad = -(-n // (NS * 8)) * (NS * 8)

    idx3 = edge_index.astype(jnp.int32).reshape(2, n_chunks, CHUNK)
    attr1 = edge_attr.astype(jnp.float32).T.reshape(
        2, 8, n_chunks, CHUNK).transpose(0, 2, 1, 3).reshape(-1).reshape(
        2 * n_chunks * 8, CHUNK)
    zeros = jnp.zeros((n_pad, d_edge), jnp.float32)

    partials = _sc_scatter_add(attr1, idx3, zeros, n_pad, n_chunks)[:, :n]
    return _tc_mlp(x, partials, W1, b1, W2, b2, ln_gamma, ln_beta,
                   block_rows=2000)


# speedup vs baseline: 9.2009x; 1.0088x over previous
import functools

import jax
import jax.numpy as jnp
from jax import lax
from jax.experimental import pallas as pl
from jax.experimental.pallas import tpu as pltpu
from jax.experimental.pallas import tpu_sc as plsc

NC = 2
NS = 16
NW = NC * NS
CHUNK = 128
SG = 20
KPT = 80


def _sc_scatter_add(attr1, idx3d, zeros, n_nodes, n_chunks):
    mesh = plsc.VectorSubcoreMesh(core_axis_name="c", subcore_axis_name="s")
    rows_per_tile = n_nodes // NS
    assert n_nodes % (NS * 8) == 0
    full_tiles = n_chunks // KPT
    rem_chunks = n_chunks - full_tiles * KPT
    sgw = SG * 8 * CHUNK

    @functools.partial(
        pl.kernel,
        mesh=mesh,
        compiler_params=pltpu.CompilerParams(
            use_tc_tiling_on_sc=False, needs_layout_passes=False),
        out_type=jax.ShapeDtypeStruct((NC, n_nodes, 16), jnp.float32),
        scratch_types=[
            pltpu.VMEM((KPT, CHUNK), jnp.int32),
            pltpu.VMEM((2 * (SG * 8 + 8), CHUNK + 1), jnp.float32),
            pltpu.VMEM((CHUNK, 16), jnp.float32),
            pltpu.VMEM_SHARED((n_nodes, 16), jnp.float32),
        ],
    )
    def sc_kernel(attr_hbm, idx_hbm, zeros_hbm, out_hbm, idx_v, raw_v,
                  tch_v, accum):
        c = lax.axis_index("c")
        s = lax.axis_index("s")
        wid = s * NC + c

        pltpu.sync_copy(
            zeros_hbm.at[pl.ds(s * rows_per_tile, rows_per_tile)],
            accum.at[pl.ds(s * rows_per_tile, rows_per_tile)],
        )
        @pl.when(wid < full_tiles)
        def _():
            pltpu.sync_copy(idx_hbm.at[1].at[pl.ds(wid * KPT, KPT)], idx_v)

        @pl.when(wid == full_tiles)
        def _():
            pltpu.sync_copy(
                idx_hbm.at[1].at[pl.ds(full_tiles * KPT, rem_chunks)],
                idx_v.at[pl.ds(0, rem_chunks)])

        plsc.subcore_barrier()

        rpp = SG * 8 + 8
        lanes = lax.iota(jnp.int32, 16)
        rvec = (lanes // 8) * rpp + (lanes % 8)

        nsg = jnp.where(wid < full_tiles, KPT // SG, rem_chunks // SG)

        @pl.loop(0, nsg)
        def _(g):
            c0 = wid * KPT + g * SG
            for tr in range(2):
                pltpu.sync_copy(
                    attr_hbm.at[pl.ds((tr * n_chunks + c0) * 8, SG * 8)],
                    raw_v.at[pl.ds(tr * rpp, SG * 8), pl.ds(0, CHUNK)])
            for j in range(SG):
                rows_j = rvec + j * 8

                @plsc.parallel_loop(0, CHUNK, step=16, unroll=4)
                def _(e0):
                    for u in range(16):
                        ev = jnp.full((16,), 0, jnp.int32) + (e0 + u)
                        row = plsc.load_gather(raw_v, [rows_j, ev])
                        tch_v[e0 + u, :] = row

                pltpu.sync_copy(
                    tch_v,
                    accum.at[idx_v.at[g * SG + j]],
                    add=True,
                )

        plsc.subcore_barrier()
        pltpu.sync_copy(
            accum.at[pl.ds(s * rows_per_tile, rows_per_tile)],
            out_hbm.at[c].at[pl.ds(s * rows_per_tile, rows_per_tile)],
        )

    return sc_kernel(attr1, idx3d, zeros)


def _tc_mlp_body(x_ref, p_ref, w1x_ref, w1a_ref, b1_ref, w2_ref, b2_ref,
                 g_ref, bt_ref, o_ref):
    x = x_ref[...]
    agg = p_ref[0] + p_ref[1]
    h = jnp.dot(x, w1x_ref[...], preferred_element_type=jnp.float32)
    h = h + jnp.dot(agg, w1a_ref[...], preferred_element_type=jnp.float32)
    h = h + b1_ref[...]
    h = h * jax.nn.sigmoid(h)
    h = jnp.dot(h, w2_ref[...], preferred_element_type=jnp.float32) + b2_ref[...]
    mean = jnp.mean(h, axis=-1, keepdims=True)
    hc = h - mean
    var = jnp.mean(hc * hc, axis=-1, keepdims=True)
    h = hc * lax.rsqrt(var + 1e-5) * g_ref[...] + bt_ref[...]
    o_ref[...] = x + h


def _tc_mlp(x, partials, W1, b1, W2, b2, ln_gamma, ln_beta, block_rows):
    n, d_feat = x.shape
    d_edge = partials.shape[-1]
    d_hid = W1.shape[1]
    d_out = W2.shape[1]
    grid = (n // block_rows,)
    full = lambda shape: pl.BlockSpec(shape, lambda i: (0,) * len(shape))
    return pl.pallas_call(
        _tc_mlp_body,
        grid=grid,
        in_specs=[
            pl.BlockSpec((block_rows, d_feat), lambda i: (i, 0)),
            pl.BlockSpec((NC, block_rows, d_edge), lambda i: (0, i, 0)),
            full((d_feat, d_hid)),
            full((d_edge, d_hid)),
            full((1, d_hid)),
            full((d_hid, d_out)),
            full((1, d_out)),
            full((1, d_out)),
            full((1, d_out)),
        ],
        out_specs=pl.BlockSpec((block_rows, d_out), lambda i: (i, 0)),
        out_shape=jax.ShapeDtypeStruct((n, d_out), jnp.float32),
    )(x, partials, W1[:d_feat], W1[d_feat:], b1.reshape(1, -1), W2,
      b2.reshape(1, -1), ln_gamma.reshape(1, -1), ln_beta.reshape(1, -1))


def kernel(x, edge_index, edge_attr, W1, b1, W2, b2, ln_gamma, ln_beta):
    n, d_feat = x.shape
    e, d_edge = edge_attr.shape

    assert d_edge == 16 and e % CHUNK == 0
    n_chunks = e // CHUNK
    assert n_chunks % SG == 0

    n_pad = -(-n // (NS * 8)) * (NS * 8)

    idx3 = edge_index.astype(jnp.int32).reshape(2, n_chunks, CHUNK)
    attr1 = edge_attr.astype(jnp.float32).T.reshape(
        2, 8, n_chunks, CHUNK).transpose(0, 2, 1, 3).reshape(-1).reshape(
        2 * n_chunks * 8, CHUNK)
    zeros = jnp.zeros((n_pad, d_edge), jnp.float32)

    partials = _sc_scatter_add(attr1, idx3, zeros, n_pad, n_chunks)[:, :n]
    return _tc_mlp(x, partials, W1, b1, W2, b2, ln_gamma, ln_beta,
                   block_rows=2000)


# speedup vs baseline: 9.8375x; 1.0692x over previous
import functools

import jax
import jax.numpy as jnp
from jax import lax
from jax.experimental import pallas as pl
from jax.experimental.pallas import tpu as pltpu
from jax.experimental.pallas import tpu_sc as plsc

NC = 2
NS = 16
NW = NC * NS
CHUNK = 128
SG = 20
KPT = 80


def _sc_scatter_add(attr1, idx3d, zeros, n_nodes, n_chunks):
    mesh = plsc.VectorSubcoreMesh(core_axis_name="c", subcore_axis_name="s")
    rows_per_tile = n_nodes // NS
    assert n_nodes % (NS * 8) == 0
    full_tiles = n_chunks // KPT
    rem_chunks = n_chunks - full_tiles * KPT
    sgw = SG * 8 * CHUNK

    @functools.partial(
        pl.kernel,
        mesh=mesh,
        compiler_params=pltpu.CompilerParams(
            use_tc_tiling_on_sc=False, needs_layout_passes=False),
        out_type=jax.ShapeDtypeStruct((NC, n_nodes, 16), jnp.float32),
        scratch_types=[
            pltpu.VMEM((KPT, CHUNK), jnp.int32),
            pltpu.VMEM((2 * (SG * 8 + 8), CHUNK + 1), jnp.float32),
            pltpu.VMEM((CHUNK, 16), jnp.float32),
            pltpu.VMEM_SHARED((n_nodes, 16), jnp.float32),
        ],
    )
    def sc_kernel(attr_hbm, idx_hbm, zeros_hbm, out_hbm, idx_v, raw_v,
                  tch_v, accum):
        c = lax.axis_index("c")
        s = lax.axis_index("s")
        wid = s * NC + c

        pltpu.sync_copy(
            zeros_hbm.at[pl.ds(s * rows_per_tile, rows_per_tile)],
            accum.at[pl.ds(s * rows_per_tile, rows_per_tile)],
        )
        @pl.when(wid < full_tiles)
        def _():
            pltpu.sync_copy(idx_hbm.at[1].at[pl.ds(wid * KPT, KPT)], idx_v)

        @pl.when(wid == full_tiles)
        def _():
            pltpu.sync_copy(
                idx_hbm.at[1].at[pl.ds(full_tiles * KPT, rem_chunks)],
                idx_v.at[pl.ds(0, rem_chunks)])

        plsc.subcore_barrier()

        rpp = SG * 8 + 8
        lanes = lax.iota(jnp.int32, 16)
        rvec = (lanes // 8) * rpp + (lanes % 8)

        nsg = jnp.where(wid < full_tiles, KPT // SG, rem_chunks // SG)

        @pl.loop(0, nsg)
        def _(g):
            c0 = wid * KPT + g * SG
            for tr in range(2):
                pltpu.sync_copy(
                    attr_hbm.at[pl.ds((tr * n_chunks + c0) * 8, SG * 8)],
                    raw_v.at[pl.ds(tr * rpp, SG * 8), pl.ds(0, CHUNK)])
            for j in range(SG):
                rows_j = rvec + j * 8

                @plsc.parallel_loop(0, CHUNK, step=16, unroll=4)
                def _(e0):
                    for u in range(16):
                        ev = jnp.full((16,), 0, jnp.int32) + (e0 + u)
                        row = plsc.load_gather(raw_v, [rows_j, ev])
                        tch_v[e0 + u, :] = row

                pltpu.sync_copy(
                    tch_v,
                    accum.at[idx_v.at[g * SG + j]],
                    add=True,
                )

        plsc.subcore_barrier()
        pltpu.sync_copy(
            accum.at[pl.ds(s * rows_per_tile, rows_per_tile)],
            out_hbm.at[c].at[pl.ds(s * rows_per_tile, rows_per_tile)],
        )

    return sc_kernel(attr1, idx3d, zeros)


def _tc_mlp_body(x_ref, p_ref, w1x_ref, w1a_ref, b1_ref, w2_ref, b2_ref,
                 g_ref, bt_ref, o_ref):
    x = x_ref[...]
    agg = p_ref[0] + p_ref[1]
    h = jnp.dot(x, w1x_ref[...], preferred_element_type=jnp.float32)
    h = h + jnp.dot(agg, w1a_ref[...], preferred_element_type=jnp.float32)
    h = h + b1_ref[...]
    h = h * jax.nn.sigmoid(h)
    h = jnp.dot(h, w2_ref[...], preferred_element_type=jnp.float32) + b2_ref[...]
    mean = jnp.mean(h, axis=-1, keepdims=True)
    hc = h - mean
    var = jnp.mean(hc * hc, axis=-1, keepdims=True)
    h = hc * lax.rsqrt(var + 1e-5) * g_ref[...] + bt_ref[...]
    o_ref[...] = x + h


def _tc_mlp(x, partials, W1, b1, W2, b2, ln_gamma, ln_beta, block_rows):
    n, d_feat = x.shape
    d_edge = partials.shape[-1]
    d_hid = W1.shape[1]
    d_out = W2.shape[1]
    grid = (n // block_rows,)
    full = lambda shape: pl.BlockSpec(shape, lambda i: (0,) * len(shape))
    return pl.pallas_call(
        _tc_mlp_body,
        grid=grid,
        in_specs=[
            pl.BlockSpec((block_rows, d_feat), lambda i: (i, 0)),
            pl.BlockSpec((NC, block_rows, d_edge), lambda i: (0, i, 0)),
            full((d_feat, d_hid)),
            full((d_edge, d_hid)),
            full((1, d_hid)),
            full((d_hid, d_out)),
            full((1, d_out)),
            full((1, d_out)),
            full((1, d_out)),
        ],
        out_specs=pl.BlockSpec((block_rows, d_out), lambda i: (i, 0)),
        out_shape=jax.ShapeDtypeStruct((n, d_out), jnp.float32),
    )(x, partials, W1[:d_feat], W1[d_feat:], b1.reshape(1, -1), W2,
      b2.reshape(1, -1), ln_gamma.reshape(1, -1), ln_beta.reshape(1, -1))


def kernel(x, edge_index, edge_attr, W1, b1, W2, b2, ln_gamma, ln_beta):
    n, d_feat = x.shape
    e, d_edge = edge_attr.shape

    assert d_edge == 16 and e % CHUNK == 0
    n_chunks = e // CHUNK
    assert n_chunks % SG == 0

    n_pad = -(-n // (NS * 8)) * (NS * 8)

    idx3 = edge_index.astype(jnp.int32).reshape(2, n_chunks, CHUNK)
    attr1 = edge_attr.astype(jnp.float32).T.reshape(
        2, 8, n_chunks, CHUNK).transpose(0, 2, 1, 3).reshape(-1).reshape(
        2 * n_chunks * 8, CHUNK)
    zeros = jnp.zeros((n_pad, d_edge), jnp.float32)

    partials = _sc_scatter_add(attr1, idx3, zeros, n_pad, n_chunks)
    return _tc_mlp(x, partials, W1, b1, W2, b2, ln_gamma, ln_beta,
                   block_rows=2000)
